# Initial kernel scaffold; baseline (speedup 1.0000x reference)
import jax
import jax.numpy as jnp
from jax.experimental import pallas as pl


def kernel(x, edge_index0, edge_index1, W, a, Ws, bs, us, Wc):
    raise NotImplementedError("write your pallas kernel here")


# speedup vs baseline: 10.7381x; 10.7381x over previous
import jax
import jax.numpy as jnp
from jax import lax
from jax.experimental import pallas as pl
from jax.experimental.pallas import tpu as pltpu
from jax.experimental.pallas import tpu_sc as plsc

N = 10000
E = 320000
NFEAT = 128
NHID = 32
NHEADS = 8
NSEM = 2
NMPATTN = 128
NLABEL = 40
ALPHA = 0.2

NCORES = 2
NSUB = 16
LANES = 16
EPT = E // NSUB
CH = 80
NCHUNK = EPT // CH
CW = 160
HHALF = NHEADS // NCORES
FH = HHALF * NHID
NPAD = 10240
ROWS_PT = NPAD // NSUB
NRS = NPAD // 8


def _proj_body(x_ref, w_ref, a_ref, h_ref, s_ref):
    h = jnp.dot(x_ref[...], w_ref[...], preferred_element_type=jnp.float32)
    h_ref[...] = h
    s_ref[...] = jnp.dot(h, a_ref[...], preferred_element_type=jnp.float32)


def _proj(x, wcat, abig):
    bn = 2000
    k = NSEM * NHEADS * NHID
    return pl.pallas_call(
        _proj_body,
        grid=(N // bn,),
        in_specs=[
            pl.BlockSpec((bn, NFEAT), lambda i: (i, 0)),
            pl.BlockSpec((NFEAT, k), lambda i: (0, 0)),
            pl.BlockSpec((k, NSEM * 16), lambda i: (0, 0)),
        ],
        out_specs=[
            pl.BlockSpec((bn, k), lambda i: (i, 0)),
            pl.BlockSpec((bn, NSEM * 16), lambda i: (i, 0)),
        ],
        out_shape=[
            jax.ShapeDtypeStruct((N, k), jnp.float32),
            jax.ShapeDtypeStruct((N, NSEM * 16), jnp.float32),
        ],
    )(x, wcat, abig)


def _vgather(vec, idx):
    dn = lax.GatherDimensionNumbers(
        offset_dims=(), collapsed_slice_dims=(0,), start_index_map=(0,))
    return lax.gather(vec, idx[:, None], dn, slice_sizes=(1,),
                      mode=lax.GatherScatterMode.PROMISE_IN_BOUNDS)


def _f32(x):
    return plsc.bitcast(x, jnp.float32)


def _sc_w_body(row_hbm, col_hbm, stp_hbm, wq_out, idxr, idxc, st_v, wbuf):
    cid = lax.axis_index("c")
    sid = lax.axis_index("s")
    coff = cid * N
    pltpu.sync_copy(stp_hbm.at[pl.ds(cid * N * 8, N * 8)], st_v)
    lane = lax.broadcasted_iota(jnp.int32, (LANES,), 0)
    zero16 = jnp.zeros((LANES,), jnp.float32)
    ebase = sid * EPT

    def _zw(i, c):
        wbuf[i] = zero16
        return c
    lax.fori_loop(0, CW, _zw, 0)

    def _chunk(i, c):
        base = ebase + i * CW
        pltpu.sync_copy(row_hbm.at[pl.ds(base, CW)], idxr)
        pltpu.sync_copy(col_hbm.at[pl.ds(base, CW)], idxc)

        def _grp(j, cc):
            rr = idxr[pl.ds(j * LANES, LANES)]
            cc16 = idxc[pl.ds(j * LANES, LANES)]
            r16 = rr * 8
            c16 = cc16 * 8
            ei = lane + j * LANES
            for t in range(HHALF):
                slv = plsc.load_gather(st_v, [r16 + t])
                dlv = plsc.load_gather(st_v, [c16 + (HHALF + t)])
                xv = slv + dlv
                wv = jnp.exp(-jnp.maximum(xv, ALPHA * xv))
                plsc.store_scatter(wbuf, [ei, jnp.full((LANES,), t, jnp.int32)], wv)
            for t, vals in ((4, cc16 + coff), (5, lax.shift_right_logical(rr, 3)),
                            (6, (rr & 7) * LANES), (7, rr)):
                plsc.store_scatter(wbuf, [ei, jnp.full((LANES,), t, jnp.int32)],
                                   _f32(vals))
            return cc
        lax.fori_loop(0, CW // LANES, _grp, 0)
        pltpu.sync_copy(wbuf, wq_out.at[cid, pl.ds(base, CW)])
        return c
    lax.fori_loop(0, EPT // CW, _chunk, 0)


def _sc_agg_body(row_hbm, col_hbm, wq_hbm, ht_hbm, hp_out, rs_out,
                 idxr, idxca, idxp, idxlb, hrows, wv_v, prod, zb,
                 hpacc, rsacc, sem3, sem4):
    cid = lax.axis_index("c")
    sid = lax.axis_index("s")
    zero16 = jnp.zeros((LANES,), jnp.float32)
    lane = lax.broadcasted_iota(jnp.int32, (LANES,), 0)
    mask4b = lane < HHALF
    mask4f = jnp.where(mask4b, 1.0, 0.0).astype(jnp.float32)
    sel4 = jnp.where(mask4b, lane, 0)
    r0 = sid * ROWS_PT
    r0rs = sid * (NRS // NSUB)

    def _z1(i, c):
        for j in range(FH // LANES):
            zb[i, pl.ds(j * LANES, LANES)] = zero16
        return c
    lax.fori_loop(0, 16, _z1, 0)

    def _zc(b, c):
        pltpu.sync_copy(zb, hpacc.at[r0 + b * LANES + lane])
        return c
    lax.fori_loop(0, ROWS_PT // LANES, _zc, 0)

    def _zr(b, c):
        pltpu.sync_copy(zb, rsacc.at[r0rs + b * LANES + lane])
        return c
    lax.fori_loop(0, NRS // NSUB // LANES, _zr, 0)
    plsc.subcore_barrier()

    coff = cid * N
    ebase = sid * EPT

    def _chunk(i, c):
        base = ebase + i * CH
        c4 = pltpu.async_copy(wq_hbm.at[cid, pl.ds(base, CH)], wv_v, sem4)
        c4.wait()
        for j in range(CH // LANES):
            jj = pl.ds(j * LANES, LANES)
            ei = lane + j * LANES
            idxca[jj] = plsc.bitcast(
                plsc.load_gather(wv_v, [ei, jnp.full((LANES,), 4, jnp.int32)]),
                jnp.int32)
            idxp[jj] = plsc.bitcast(
                plsc.load_gather(wv_v, [ei, jnp.full((LANES,), 5, jnp.int32)]),
                jnp.int32)
            idxlb[jj] = plsc.bitcast(
                plsc.load_gather(wv_v, [ei, jnp.full((LANES,), 6, jnp.int32)]),
                jnp.int32)
            idxr[jj] = plsc.bitcast(
                plsc.load_gather(wv_v, [ei, jnp.full((LANES,), 7, jnp.int32)]),
                jnp.int32)
        c3 = pltpu.async_copy(ht_hbm.at[idxca], hrows, sem3)

        def _zp(e, cc):
            for k in range(FH // LANES):
                prod[e, pl.ds(k * LANES, LANES)] = zero16
            return cc
        lax.fori_loop(0, CH, _zp, 0)

        def _edgew(e, cc):
            ev = jnp.zeros((LANES,), jnp.int32) + e
            w = wv_v[e] * mask4f
            lb = plsc.load_gather(idxlb, [ev])
            plsc.store_scatter(prod, [ev, lb + sel4], w, mask=mask4b)
            return cc
        lax.fori_loop(0, CH, _edgew, 0)
        pltpu.sync_copy(prod, rsacc.at[idxp], add=True)

        c3.wait()

        def _edge(e, cc):
            w = wv_v[e]
            for k in range(FH // LANES):
                hk = hrows[e, pl.ds(k * LANES, LANES)]
                wk = _vgather(w, jnp.full((LANES,), k // 2, jnp.int32))
                prod[e, pl.ds(k * LANES, LANES)] = hk * wk
            return cc
        lax.fori_loop(0, CH, _edge, 0)
        pltpu.sync_copy(prod, hpacc.at[idxr], add=True)
        return c
    lax.fori_loop(0, NCHUNK, _chunk, 0)
    plsc.subcore_barrier()

    def _wo(b, c):
        pltpu.sync_copy(hpacc.at[r0 + b * LANES + lane], zb)
        pltpu.sync_copy(zb, hp_out.at[cid, pl.ds(r0 + b * LANES, LANES)])
        return c
    lax.fori_loop(0, ROWS_PT // LANES, _wo, 0)

    def _wor(b, c):
        pltpu.sync_copy(rsacc.at[r0rs + b * LANES + lane], zb)
        pltpu.sync_copy(zb, rs_out.at[cid, pl.ds(r0rs + b * LANES, LANES)])
        return c
    lax.fori_loop(0, NRS // NSUB // LANES, _wor, 0)


def _sc_w(row, col, stp):
    mesh = plsc.VectorSubcoreMesh(core_axis_name="c", subcore_axis_name="s")
    params = pltpu.CompilerParams(needs_layout_passes=False)
    return pl.kernel(
        _sc_w_body,
        out_type=jax.ShapeDtypeStruct((NCORES, E, LANES), jnp.float32),
        mesh=mesh,
        compiler_params=params,
        scratch_types=[
            pltpu.VMEM((CW,), jnp.int32),
            pltpu.VMEM((CW,), jnp.int32),
            pltpu.VMEM((N * 8,), jnp.float32),
            pltpu.VMEM((CW, LANES), jnp.float32),
        ],
    )(row, col, stp)


def _sc_main(row, col, wq, ht):
    mesh = plsc.VectorSubcoreMesh(core_axis_name="c", subcore_axis_name="s")
    params = pltpu.CompilerParams(needs_layout_passes=False)
    f = pl.kernel(
        _sc_agg_body,
        out_type=[
            jax.ShapeDtypeStruct((NCORES, NPAD, FH), jnp.float32),
            jax.ShapeDtypeStruct((NCORES, NRS, FH), jnp.float32),
        ],
        mesh=mesh,
        compiler_params=params,
        scratch_types=[
            pltpu.VMEM((CH,), jnp.int32),
            pltpu.VMEM((CH,), jnp.int32),
            pltpu.VMEM((CH,), jnp.int32),
            pltpu.VMEM((CH,), jnp.int32),
            pltpu.VMEM((CH, FH), jnp.float32),
            pltpu.VMEM((CH, LANES), jnp.float32),
            pltpu.VMEM((CH, FH), jnp.float32),
            pltpu.VMEM((16, FH), jnp.float32),
            pltpu.VMEM_SHARED((NPAD, FH), jnp.float32),
            pltpu.VMEM_SHARED((NRS, FH), jnp.float32),
            pltpu.SemaphoreType.DMA,
            pltpu.SemaphoreType.DMA,
        ],
    )
    return f(row, col, wq, ht)


def _sc_agg(row, col, stp, ht):
    wq = _sc_w(row, col, stp)
    return _sc_main(row, col, wq, ht)


def _elu(t):
    return jnp.where(t > 0, t, jnp.exp(jnp.minimum(t, 0.0)) - 1.0)


def _epi_body(s0_ref, r0_ref, s1_ref, r1_ref, ws_ref, bs_ref, us_ref, wc_ref,
              o_ref):
    e0 = _elu(s0_ref[...] / (r0_ref[...] + 1e-16))
    e1 = _elu(s1_ref[...] / (r1_ref[...] + 1e-16))
    v0 = jnp.tanh(jnp.dot(e0, ws_ref[...], preferred_element_type=jnp.float32)
                  + bs_ref[...])
    v1 = jnp.tanh(jnp.dot(e1, ws_ref[...], preferred_element_type=jnp.float32)
                  + bs_ref[...])
    u = us_ref[...]
    vu0 = jnp.sum(v0 * u, axis=1, keepdims=True)
    vu1 = jnp.sum(v1 * u, axis=1, keepdims=True)
    m = jnp.maximum(vu0, vu1)
    x0 = jnp.exp(vu0 - m)
    x1 = jnp.exp(vu1 - m)
    inv = 1.0 / (x0 + x1)
    fin = (x0 * inv) * e0 + (x1 * inv) * e1
    logits = jnp.dot(fin, wc_ref[...], preferred_element_type=jnp.float32)
    o_ref[...] = 1.0 / (1.0 + jnp.exp(-logits))


def _epi(s0, rse0, s1, rse1, Ws, bs2, us2, wct):
    bn = 2000
    d = NHEADS * NHID
    return pl.pallas_call(
        _epi_body,
        grid=(N // bn,),
        in_specs=[
            pl.BlockSpec((bn, d), lambda i: (i, 0)),
            pl.BlockSpec((bn, d), lambda i: (i, 0)),
            pl.BlockSpec((bn, d), lambda i: (i, 0)),
            pl.BlockSpec((bn, d), lambda i: (i, 0)),
            pl.BlockSpec((d, NMPATTN), lambda i: (0, 0)),
            pl.BlockSpec((1, NMPATTN), lambda i: (0, 0)),
            pl.BlockSpec((1, NMPATTN), lambda i: (0, 0)),
            pl.BlockSpec((d, NLABEL), lambda i: (0, 0)),
        ],
        out_specs=pl.BlockSpec((bn, NLABEL), lambda i: (i, 0)),
        out_shape=jax.ShapeDtypeStruct((N, NLABEL), jnp.float32),
    )(s0, rse0, s1, rse1, Ws, bs2, us2, wct)


def kernel(x, edge_index0, edge_index1, W, a, Ws, bs, us, Wc):
    k = NSEM * NHEADS * NHID
    wcat = jnp.transpose(W, (2, 0, 1, 3)).reshape(NFEAT, k)
    eye2 = jnp.eye(NSEM, dtype=jnp.float32)
    eye8 = jnp.eye(NHEADS, dtype=jnp.float32)
    a_src = a[:, :, :NHID]
    a_dst = a[:, :, NHID:]
    A_s = jnp.einsum("shj,sk,ht->shjkt", a_src, eye2, eye8)
    A_d = jnp.einsum("shj,sk,ht->shjkt", a_dst, eye2, eye8)
    abig = jnp.concatenate([A_s, A_d], axis=4).reshape(k, NSEM * 16)

    h_both, scores = _proj(x, wcat, abig)

    embs, rses = [], []
    for s, ei in ((0, edge_index0), (1, edge_index1)):
        d = NHEADS * NHID
        ht = (h_both[:, s * d:(s + 1) * d]
              .reshape(N, NCORES, FH).swapaxes(0, 1).reshape(NCORES * N, FH))
        sl = scores[:, s * 16:s * 16 + 8]
        dl = scores[:, s * 16 + 8:s * 16 + 16]
        stp = jnp.stack([
            jnp.concatenate([sl[:, :HHALF], dl[:, :HHALF]], axis=1),
            jnp.concatenate([sl[:, HHALF:], dl[:, HHALF:]], axis=1),
        ]).reshape(NCORES * N * 8)
        hp, rs = _sc_agg(ei[0], ei[1], stp, ht)
        embs.append(hp[:, :N].swapaxes(0, 1).reshape(N, NCORES * FH))
        rsp = rs.reshape(NCORES, NRS, 8, LANES)[:, :, :, :HHALF]
        rsum = (rsp.reshape(NCORES, NPAD, HHALF)[:, :N]
                .swapaxes(0, 1).reshape(N, NHEADS))
        rses.append(jnp.repeat(rsum, NHID, axis=1))

    return _epi(embs[0], rses[0], embs[1], rses[1],
                Ws, bs.reshape(1, NMPATTN), us.reshape(1, NMPATTN), Wc.T)
